# baseline (speedup 1.0000x reference)
import functools

import jax
import jax.numpy as jnp
from jax import lax
from jax.experimental import pallas as pl
from jax.experimental.pallas import tpu as pltpu
from jax.experimental.pallas import tpu_sc as plsc

B = 16384
D = 64
C = 1000
C1 = 1024
MARGIN = 0.3
EPS = 1e-12

NC = 2
NS = 16
NW = NC * NS
RPW = B // NW
NCHUNK = RPW // 64


def _normalize_body(emb_ref, out_ref):
    e = emb_ref[...]
    ss = jnp.sum(e * e, axis=1, keepdims=True)
    nrm = jnp.maximum(jnp.sqrt(ss), EPS)
    out_ref[...] = e / nrm


def _segsum_body(en_hbm, lab_hbm, z64_hbm, z16_hbm, sums_hbm, counts_hbm,
                 en_v, lab_v, ones_v, ssum, scnt, sem):
    cid = lax.axis_index("c")
    sid = lax.axis_index("s")
    wid = cid * NS + sid
    base = wid * RPW
    rows = C1 // NS

    h0 = pltpu.async_copy(z64_hbm.at[pl.ds(sid * rows, rows)],
                          ssum.at[pl.ds(sid * rows, rows)], sem)
    h1 = pltpu.async_copy(z16_hbm.at[pl.ds(sid * rows, rows)],
                          scnt.at[pl.ds(sid * rows, rows)], sem)
    h2 = pltpu.async_copy(en_hbm.at[pl.ds(base, RPW)], en_v, sem)
    h3 = pltpu.async_copy(lab_hbm.at[wid], lab_v, sem)
    one_row = jnp.full((16,), 1.0, jnp.float32)

    @pl.loop(0, RPW)
    def _(i):
        ones_v[i, :] = one_row

    h0.wait()
    h1.wait()
    h2.wait()
    h3.wait()
    plsc.subcore_barrier()

    hs = []
    for j in range(NCHUNK):
        idx = lab_v.at[j, 0]
        hs.append(pltpu.async_copy(en_v.at[pl.ds(j * 64, 64)],
                                   ssum.at[idx], sem, add=True))
        hs.append(pltpu.async_copy(ones_v.at[pl.ds(j * 64, 64)],
                                   scnt.at[idx], sem, add=True))
    for h in hs:
        h.wait()
    plsc.subcore_barrier()

    pltpu.sync_copy(ssum.at[pl.ds(sid * rows, rows)],
                    sums_hbm.at[cid, pl.ds(sid * rows, rows)])
    pltpu.sync_copy(scnt.at[pl.ds(sid * rows, rows)],
                    counts_hbm.at[cid, pl.ds(sid * rows, rows)])


def _centroid_body(sums_ref, counts_ref, w_ref):
    sums = sums_ref[0, :, :] + sums_ref[1, :, :]
    counts = counts_ref[0, :, 0:1] + counts_ref[1, :, 0:1]
    safe = jnp.maximum(counts, 1.0)
    cen = sums / safe
    cn = jnp.maximum(jnp.sqrt(jnp.sum(cen * cen, axis=1, keepdims=True)), EPS)
    cen = cen / cn

    cen2 = cen * cen
    sq_col = jnp.sum(cen2, axis=1, keepdims=True)
    ones_row = jnp.ones((1, D), jnp.float32)
    sq_row = lax.dot_general(ones_row, cen2, (((1,), (1,)), ((), ())),
                             preferred_element_type=jnp.float32)
    g = lax.dot_general(cen, cen, (((1,), (1,)), ((), ())),
                        preferred_element_type=jnp.float32)
    d2 = jnp.maximum(sq_col + sq_row - 2.0 * g, 0.0)
    dist = jnp.sqrt(d2)
    row_i = lax.broadcasted_iota(jnp.int32, (C1, C1), 0)
    col_i = lax.broadcasted_iota(jnp.int32, (C1, C1), 1)
    dist = jnp.where((row_i == col_i) | (col_i >= C), jnp.inf, dist)
    minv = jnp.min(dist, axis=1, keepdims=True)
    nearest = jnp.min(jnp.where(dist == minv, col_i, jnp.int32(2 ** 30)),
                      axis=1, keepdims=True)

    oh_n = (col_i == nearest).astype(jnp.float32)
    c_neg = lax.dot_general(oh_n, cen, (((1,), (0,)), ((), ())),
                            preferred_element_type=jnp.float32)
    w_ref[...] = c_neg - cen


def _vals_body(en_hbm, lab_hbm, w_hbm, z16_hbm, vsum_hbm,
               en_v, w_rows, lab_v, idx_v, acc, svs, sem, sem_idx):
    cid = lax.axis_index("c")
    sid = lax.axis_index("s")
    wid = cid * NS + sid
    base = wid * RPW
    rows = C1 // NS

    h0 = pltpu.async_copy(z16_hbm.at[pl.ds(sid * rows, rows)],
                          svs.at[pl.ds(sid * rows, rows)], sem)
    h1 = pltpu.async_copy(en_hbm.at[pl.ds(base, RPW)], en_v, sem)
    h2 = pltpu.async_copy(lab_hbm.at[wid], lab_v, sem_idx)
    h4 = pltpu.async_copy(z16_hbm, acc, sem)
    h2.wait()

    hs = []
    for j in range(NCHUNK):
        hs.append(pltpu.async_copy(w_hbm.at[lab_v.at[j, 0]],
                                   w_rows.at[pl.ds(j * 64, 64)], sem))
    h0.wait()
    h1.wait()
    h4.wait()
    for h in hs:
        h.wait()

    iota16 = lax.iota(jnp.int32, 16)
    lane0 = (iota16 == 0).astype(jnp.float32)

    @pl.loop(0, RPW // 16)
    def _(g):
        j = g // 4
        o = (g % 4) * 16
        lv = lab_v[j, 0, pl.ds(o, 16)]
        for k in range(16):
            i = g * 16 + k
            p = en_v[i, pl.ds(0, 16)] * w_rows[i, pl.ds(0, 16)]
            p = p + en_v[i, pl.ds(16, 16)] * w_rows[i, pl.ds(16, 16)]
            p = p + en_v[i, pl.ds(32, 16)] * w_rows[i, pl.ds(32, 16)]
            p = p + en_v[i, pl.ds(48, 16)] * w_rows[i, pl.ds(48, 16)]
            t = jnp.sum(p)
            v = jnp.maximum(t + MARGIN, 0.0)
            l = lv[k]
            acc[l, :] += v * lane0

    @pl.loop(0, C1 // 64)
    def _(i):
        b = i * 64
        for q in range(4):
            idx_v[i, 0, pl.ds(q * 16, 16)] = iota16 + (b + q * 16)

    plsc.subcore_barrier()
    hs = []
    for j in range(C1 // 64):
        hs.append(pltpu.async_copy(acc.at[pl.ds(j * 64, 64)],
                                   svs.at[idx_v.at[j, 0]], sem, add=True))
    for h in hs:
        h.wait()
    plsc.subcore_barrier()

    pltpu.sync_copy(svs.at[pl.ds(sid * rows, rows)],
                    vsum_hbm.at[cid, pl.ds(sid * rows, rows)])


def _finalize_body(vsum_ref, counts_ref, out_ref):
    vsum = vsum_ref[0, :, 0:1] + vsum_ref[1, :, 0:1]
    counts = counts_ref[0, :, 0:1] + counts_ref[1, :, 0:1]
    safe = jnp.maximum(counts, 1.0)
    per_class = vsum / safe
    present = (counts > 0.0).astype(jnp.float32)
    num = jnp.sum(per_class * present, axis=0, keepdims=True)
    den = jnp.maximum(jnp.sum(present, axis=0, keepdims=True), 1.0)
    out_ref[...] = num / den


_SC_MESH = plsc.VectorSubcoreMesh(core_axis_name="c", subcore_axis_name="s")
_SC_PARAMS = pltpu.CompilerParams(needs_layout_passes=False,
                                  use_tc_tiling_on_sc=False)

_segsum_call = functools.partial(
    pl.kernel, _segsum_body,
    out_type=(jax.ShapeDtypeStruct((NC, C1, D), jnp.float32),
              jax.ShapeDtypeStruct((NC, C1, 16), jnp.float32)),
    mesh=_SC_MESH,
    scratch_types=[
        pltpu.VMEM((RPW, D), jnp.float32),
        pltpu.VMEM((NCHUNK, 1, 64), jnp.int32),
        pltpu.VMEM((RPW, 16), jnp.float32),
        pltpu.VMEM_SHARED((C1, D), jnp.float32),
        pltpu.VMEM_SHARED((C1, 16), jnp.float32),
        pltpu.SemaphoreType.DMA,
    ],
    compiler_params=_SC_PARAMS,
)()

_vals_call = functools.partial(
    pl.kernel, _vals_body,
    out_type=jax.ShapeDtypeStruct((NC, C1, 16), jnp.float32),
    mesh=_SC_MESH,
    scratch_types=[
        pltpu.VMEM((RPW, D), jnp.float32),
        pltpu.VMEM((RPW, D), jnp.float32),
        pltpu.VMEM((NCHUNK, 1, 64), jnp.int32),
        pltpu.VMEM((C1 // 64, 1, 64), jnp.int32),
        pltpu.VMEM((C1, 16), jnp.float32),
        pltpu.VMEM_SHARED((C1, 16), jnp.float32),
        pltpu.SemaphoreType.DMA,
        pltpu.SemaphoreType.DMA,
    ],
    compiler_params=_SC_PARAMS,
)()


def kernel(embeddings, labels):
    lab_chunks = labels.reshape(NW, NCHUNK, 1, 64)
    z64 = jnp.zeros((C1, D), jnp.float32)
    z16 = jnp.zeros((C1, 16), jnp.float32)

    en = pl.pallas_call(
        _normalize_body,
        out_shape=jax.ShapeDtypeStruct((B, D), jnp.float32),
    )(embeddings)

    sums, counts = _segsum_call(en, lab_chunks, z64, z16)

    w = pl.pallas_call(
        _centroid_body,
        out_shape=jax.ShapeDtypeStruct((C1, D), jnp.float32),
    )(sums, counts)

    vsum = _vals_call(en, lab_chunks, w, z16)

    out = pl.pallas_call(
        _finalize_body,
        out_shape=jax.ShapeDtypeStruct((1, 1), jnp.float32),
    )(vsum, counts)
    return out[0, 0]


# speedup vs baseline: 1.2393x; 1.2393x over previous
import functools

import jax
import jax.numpy as jnp
from jax import lax
from jax.experimental import pallas as pl
from jax.experimental.pallas import tpu as pltpu
from jax.experimental.pallas import tpu_sc as plsc

B = 16384
D = 64
C = 1000
C1 = 1024
TW = 80
MARGIN = 0.3
EPS = 1e-12

NC = 2
NS = 16
NW = NC * NS
RPW = B // NW
NCHUNK = RPW // 64

_SC_MESH = plsc.VectorSubcoreMesh(core_axis_name="c", subcore_axis_name="s")
_SC_PARAMS = pltpu.CompilerParams(needs_layout_passes=False,
                                  use_tc_tiling_on_sc=False)


def _rsqrt_scale(ss):
    bi = lax.bitcast_convert_type(ss, jnp.int32)
    yi = jnp.int32(0x5F3759DF) - lax.shift_right_arithmetic(bi, jnp.int32(1))
    y = lax.bitcast_convert_type(yi, jnp.float32)
    for _ in range(3):
        y = y * (1.5 - 0.5 * ss * y * y)
    return jnp.minimum(y, 1e12)


def _segsum_body(emb_hbm, lab_hbm, z64_hbm, z16_hbm, sums_hbm, counts_hbm,
                 en_hbm, en_v, lab_v, ones_v, ssum, scnt, sem):
    cid = lax.axis_index("c")
    sid = lax.axis_index("s")
    wid = cid * NS + sid
    base = wid * RPW
    rows = C1 // NS

    one_row = jnp.full((16,), 1.0, jnp.float32)

    @pl.loop(0, RPW)
    def _(i):
        ones_v[i, :] = one_row

    hs = [pltpu.async_copy(z64_hbm.at[pl.ds(sid * rows, rows)],
                           ssum.at[pl.ds(sid * rows, rows)], sem),
          pltpu.async_copy(z16_hbm.at[pl.ds(sid * rows, rows)],
                           scnt.at[pl.ds(sid * rows, rows)], sem),
          pltpu.async_copy(emb_hbm.at[pl.ds(base, RPW)], en_v, sem)]
    hs.append(pltpu.async_copy(lab_hbm.at[wid], lab_v, sem))
    for h in hs:
        h.wait()

    @pl.loop(0, RPW // 16)
    def _(g):
        for k in range(16):
            i = g * 16 + k
            e0 = en_v[i, pl.ds(0, 16)]
            e1 = en_v[i, pl.ds(16, 16)]
            e2 = en_v[i, pl.ds(32, 16)]
            e3 = en_v[i, pl.ds(48, 16)]
            p = e0 * e0 + e1 * e1 + e2 * e2 + e3 * e3
            sc = _rsqrt_scale(jnp.sum(p))
            en_v[i, pl.ds(0, 16)] = e0 * sc
            en_v[i, pl.ds(16, 16)] = e1 * sc
            en_v[i, pl.ds(32, 16)] = e2 * sc
            en_v[i, pl.ds(48, 16)] = e3 * sc

    plsc.subcore_barrier()

    hs = []
    for j in range(NCHUNK):
        idx = lab_v.at[j, 0]
        hs.append(pltpu.async_copy(en_v.at[pl.ds(j * 64, 64)],
                                   ssum.at[idx], sem, add=True))
        hs.append(pltpu.async_copy(ones_v.at[pl.ds(j * 64, 64)],
                                   scnt.at[idx], sem, add=True))
    for h in hs:
        h.wait()
    pltpu.sync_copy(en_v, en_hbm.at[pl.ds(base, RPW)])
    plsc.subcore_barrier()

    pltpu.sync_copy(ssum.at[pl.ds(sid * rows, rows)],
                    sums_hbm.at[cid, pl.ds(sid * rows, rows)])
    pltpu.sync_copy(scnt.at[pl.ds(sid * rows, rows)],
                    counts_hbm.at[cid, pl.ds(sid * rows, rows)])


_segsum_call = functools.partial(
    pl.kernel, _segsum_body,
    out_type=(jax.ShapeDtypeStruct((NC, C1, D), jnp.float32),
              jax.ShapeDtypeStruct((NC, C1, 16), jnp.float32),
              jax.ShapeDtypeStruct((B, D), jnp.float32)),
    mesh=_SC_MESH,
    scratch_types=[
        pltpu.VMEM((RPW, D), jnp.float32),
        pltpu.VMEM((NCHUNK, 1, 64), jnp.int32),
        pltpu.VMEM((RPW, 16), jnp.float32),
        pltpu.VMEM_SHARED((C1, D), jnp.float32),
        pltpu.VMEM_SHARED((C1, 16), jnp.float32),
        pltpu.SemaphoreType.DMA,
    ],
    compiler_params=_SC_PARAMS,
)()


def _centroid_body(sums_ref, counts_ref, t_ref):
    sums = sums_ref[0, :, :] + sums_ref[1, :, :]
    counts = counts_ref[0, :, 0:1] + counts_ref[1, :, 0:1]
    safe = jnp.maximum(counts, 1.0)
    cen = sums / safe
    cn = jnp.maximum(jnp.sqrt(jnp.sum(cen * cen, axis=1, keepdims=True)), EPS)
    cen = cen / cn

    cen2 = cen * cen
    sq_col = jnp.sum(cen2, axis=1, keepdims=True)
    ones_row = jnp.ones((1, D), jnp.float32)
    sq_row = lax.dot_general(ones_row, cen2, (((1,), (1,)), ((), ())),
                             preferred_element_type=jnp.float32)
    g = lax.dot_general(cen, cen, (((1,), (1,)), ((), ())),
                        preferred_element_type=jnp.float32)
    d2 = jnp.maximum(sq_col + sq_row - 2.0 * g, 0.0)
    dist = jnp.sqrt(d2)
    row_i = lax.broadcasted_iota(jnp.int32, (C1, C1), 0)
    col_i = lax.broadcasted_iota(jnp.int32, (C1, C1), 1)
    dist = jnp.where((row_i == col_i) | (col_i >= C), jnp.inf, dist)
    minv = jnp.min(dist, axis=1, keepdims=True)
    nearest = jnp.min(jnp.where(dist == minv, col_i, jnp.int32(2 ** 30)),
                      axis=1, keepdims=True)

    oh_n = (col_i == nearest).astype(jnp.float32)
    c_neg = lax.dot_general(oh_n, cen, (((1,), (0,)), ((), ())),
                            preferred_element_type=jnp.float32)
    t_ref[:, 0:D] = c_neg - cen

    present = (counts > 0.0).astype(jnp.float32)
    den = jnp.maximum(jnp.sum(present, axis=0, keepdims=True), 1.0)
    s = present / (safe * den)
    lane0 = (lax.broadcasted_iota(jnp.int32, (C1, 16), 1) == 0)
    t_ref[:, D:TW] = s * lane0.astype(jnp.float32)


def _vals_body(en_hbm, lab_hbm, t_hbm, out_hbm,
               en_v, t_rows, lab_v, acc_v, sem, sem_idx):
    cid = lax.axis_index("c")
    sid = lax.axis_index("s")
    wid = cid * NS + sid
    base = wid * RPW

    h1 = pltpu.async_copy(en_hbm.at[pl.ds(base, RPW)], en_v, sem)
    h2 = pltpu.async_copy(lab_hbm.at[wid], lab_v, sem_idx)
    h2.wait()

    hs = [pltpu.async_copy(t_hbm.at[lab_v.at[j, 0]],
                           t_rows.at[pl.ds(j * 64, 64)], sem)
          for j in range(NCHUNK)]
    h1.wait()
    for h in hs:
        h.wait()

    def body(g, acc):
        for k in range(16):
            i = g * 16 + k
            p = en_v[i, pl.ds(0, 16)] * t_rows[i, pl.ds(0, 16)]
            p = p + en_v[i, pl.ds(16, 16)] * t_rows[i, pl.ds(16, 16)]
            p = p + en_v[i, pl.ds(32, 16)] * t_rows[i, pl.ds(32, 16)]
            p = p + en_v[i, pl.ds(48, 16)] * t_rows[i, pl.ds(48, 16)]
            t = jnp.sum(p)
            v = jnp.maximum(t + MARGIN, 0.0)
            acc = acc + v * t_rows[i, pl.ds(D, 16)]
        return acc

    acc = lax.fori_loop(0, RPW // 16, body, jnp.zeros((16,), jnp.float32))
    acc_v[...] = acc
    pltpu.sync_copy(acc_v, out_hbm.at[cid, sid])


_vals_call = functools.partial(
    pl.kernel, _vals_body,
    out_type=jax.ShapeDtypeStruct((NC, NS, 16), jnp.float32),
    mesh=_SC_MESH,
    scratch_types=[
        pltpu.VMEM((RPW, D), jnp.float32),
        pltpu.VMEM((RPW, TW), jnp.float32),
        pltpu.VMEM((NCHUNK, 1, 64), jnp.int32),
        pltpu.VMEM((16,), jnp.float32),
        pltpu.SemaphoreType.DMA,
        pltpu.SemaphoreType.DMA,
    ],
    compiler_params=_SC_PARAMS,
)()


def _finalize_body(part_ref, out_ref):
    p = part_ref[0, :, :] + part_ref[1, :, :]
    num = jnp.sum(p, axis=0, keepdims=True)
    out_ref[...] = jnp.sum(num, axis=1, keepdims=True)


def kernel(embeddings, labels):
    z64 = jnp.zeros((C1, D), jnp.float32)
    z16 = jnp.zeros((C1, 16), jnp.float32)
    lab_chunks = labels.reshape(NW, NCHUNK, 1, 64)
    sums, counts, en = _segsum_call(embeddings, lab_chunks, z64, z16)

    t = pl.pallas_call(
        _centroid_body,
        out_shape=jax.ShapeDtypeStruct((C1, TW), jnp.float32),
    )(sums, counts)

    part = _vals_call(en, lab_chunks, t)

    out = pl.pallas_call(
        _finalize_body,
        out_shape=jax.ShapeDtypeStruct((1, 1), jnp.float32),
    )(part)
    return out[0, 0]
